# Initial kernel scaffold; baseline (speedup 1.0000x reference)
import jax
import jax.numpy as jnp
from jax.experimental import pallas as pl


def kernel(mlp_input, mlp_output, W_enc, b_enc, W_dec, b_dec):
    raise NotImplementedError("write your pallas kernel here")


# speedup vs baseline: 9.5435x; 9.5435x over previous
import jax
import jax.numpy as jnp
from jax.experimental import pallas as pl
from jax.experimental.pallas import tpu as pltpu

K_TOP = 64
CH = 128
R_CAND = 8
BR1 = 512
BH1 = 2048
BR2 = 128
BR3 = 1024
BK3 = 1024


def _enc_kernel(x_ref, w_ref, b_ref, o_ref):
    acc = jax.lax.dot_general(
        x_ref[...], w_ref[...], (((1,), (1,)), ((), ())),
        preferred_element_type=jnp.float32)
    o_ref[...] = acc + b_ref[...]


def _topk_kernel(x_ref, h_ref, l0_ref, y_ref, cand_ref):
    nch = x_ref.shape[1]
    x = x_ref[...]
    y_ref[...] = x
    neg = jnp.float32(-jnp.inf)
    for r in range(R_CAND):
        y = y_ref[...]
        cm = jnp.max(y, axis=2)
        cand_ref[:, r * nch:(r + 1) * nch] = cm
        y_ref[...] = jnp.where(y == cm[:, :, None], neg, y)

    def sel_body(i, m):
        c = cand_ref[...]
        m = jnp.max(c, axis=1, keepdims=True)
        cand_ref[...] = jnp.where(c == m, neg, c)
        return m

    t = jax.lax.fori_loop(0, K_TOP, sel_body,
                          jnp.zeros((x.shape[0], 1), jnp.float32))
    t3 = t[:, :, None]
    cnt = jnp.sum(jnp.sum((x >= t3).astype(jnp.float32), axis=2),
                  axis=1, keepdims=True)
    bad = cnt != jnp.float32(K_TOP)

    y_ref[...] = x

    def fb_body(i, m):
        yy = y_ref[...]
        m = jnp.max(jnp.max(yy, axis=2), axis=1, keepdims=True)
        y_ref[...] = jnp.where(yy == m[:, :, None], neg, yy)
        return m

    @pl.when(jnp.any(bad))
    def _():
        t2 = jax.lax.fori_loop(0, K_TOP, fb_body,
                               jnp.zeros((x.shape[0], 1), jnp.float32))
        cand_ref[:, 0:1] = jnp.where(bad, t2, t)

    @pl.when(jnp.logical_not(jnp.any(bad)))
    def _():
        cand_ref[:, 0:1] = t

    tf = cand_ref[:, 0:1][:, :, None]
    mask = x >= tf
    h_ref[...] = jnp.where(mask, jnp.maximum(x, 0.0), 0.0)
    l0_ref[...] = jnp.sum(
        jnp.sum(jnp.logical_and(mask, x > 0).astype(jnp.float32), axis=2),
        axis=1, keepdims=True)


def _dec_kernel(h_ref, w_ref, b_ref, y_ref, o_ref, lr_ref, *, nk):
    k = pl.program_id(1)

    @pl.when(k == 0)
    def _():
        o_ref[...] = jnp.zeros_like(o_ref)

    o_ref[...] += jax.lax.dot_general(
        h_ref[...], w_ref[...], (((1,), (1,)), ((), ())),
        preferred_element_type=jnp.float32)

    @pl.when(k == nk - 1)
    def _():
        pred = o_ref[...] + b_ref[...]
        o_ref[...] = pred
        d = pred - y_ref[...]
        lr_ref[...] = jnp.sum(d * d, axis=1, keepdims=True)


def kernel(mlp_input, mlp_output, W_enc, b_enc, W_dec, b_dec):
    n_tok, d_in = mlp_input.shape
    d_hid = W_enc.shape[0]
    d_out = W_dec.shape[0]
    nch = d_hid // CH

    br1 = min(BR1, n_tok)
    bh1 = min(BH1, d_hid)
    pre = pl.pallas_call(
        _enc_kernel,
        grid=(d_hid // bh1, n_tok // br1),
        in_specs=[
            pl.BlockSpec((br1, d_in), lambda h, r: (r, 0)),
            pl.BlockSpec((bh1, d_in), lambda h, r: (h, 0)),
            pl.BlockSpec((1, bh1), lambda h, r: (0, h)),
        ],
        out_specs=pl.BlockSpec((br1, bh1), lambda h, r: (r, h)),
        out_shape=jax.ShapeDtypeStruct((n_tok, d_hid), jnp.float32),
        compiler_params=pltpu.CompilerParams(
            dimension_semantics=("arbitrary", "arbitrary")),
    )(mlp_input, W_enc, b_enc.reshape(1, d_hid))

    pre3 = pre.reshape(n_tok, nch, CH)
    br2 = min(BR2, n_tok)
    hidden3, l0_rows = pl.pallas_call(
        _topk_kernel,
        grid=(n_tok // br2,),
        in_specs=[pl.BlockSpec((br2, nch, CH), lambda i: (i, 0, 0))],
        out_specs=[
            pl.BlockSpec((br2, nch, CH), lambda i: (i, 0, 0)),
            pl.BlockSpec((br2, 1), lambda i: (i, 0)),
        ],
        out_shape=[
            jax.ShapeDtypeStruct((n_tok, nch, CH), jnp.float32),
            jax.ShapeDtypeStruct((n_tok, 1), jnp.float32),
        ],
        scratch_shapes=[
            pltpu.VMEM((br2, nch, CH), jnp.float32),
            pltpu.VMEM((br2, nch * R_CAND), jnp.float32),
        ],
        compiler_params=pltpu.CompilerParams(
            dimension_semantics=("arbitrary",)),
    )(pre3)
    hidden = hidden3.reshape(n_tok, d_hid)

    br3 = min(BR3, n_tok)
    bk3 = min(BK3, d_hid)
    nk = d_hid // bk3
    import functools
    predicted, loss_rows = pl.pallas_call(
        functools.partial(_dec_kernel, nk=nk),
        grid=(n_tok // br3, nk),
        in_specs=[
            pl.BlockSpec((br3, bk3), lambda r, k: (r, k)),
            pl.BlockSpec((d_out, bk3), lambda r, k: (0, k)),
            pl.BlockSpec((1, d_out), lambda r, k: (0, 0)),
            pl.BlockSpec((br3, d_out), lambda r, k: (r, 0)),
        ],
        out_specs=[
            pl.BlockSpec((br3, d_out), lambda r, k: (r, 0)),
            pl.BlockSpec((br3, 1), lambda r, k: (r, 0)),
        ],
        out_shape=[
            jax.ShapeDtypeStruct((n_tok, d_out), jnp.float32),
            jax.ShapeDtypeStruct((n_tok, 1), jnp.float32),
        ],
        compiler_params=pltpu.CompilerParams(
            dimension_semantics=("arbitrary", "arbitrary")),
    )(hidden, W_dec, b_dec.reshape(1, d_out), mlp_output)

    reconstruction_loss = jnp.sum(loss_rows) / jnp.float32(n_tok * d_out)
    l0 = jnp.sum(l0_rows) / jnp.float32(n_tok)
    sparsity_loss = jnp.asarray(0.0, dtype=jnp.float32)
    loss = reconstruction_loss
    return (predicted, hidden, loss, reconstruction_loss, sparsity_loss, l0)


# speedup vs baseline: 12.0559x; 1.2633x over previous
import jax
import jax.numpy as jnp
from jax.experimental import pallas as pl
from jax.experimental.pallas import tpu as pltpu

K_TOP = 64
CH = 128
R_CAND = 8
BR1 = 512
BH1 = 2048
BR2 = 128
BR3 = 1024
BK3 = 1024


def _enc_kernel(x_ref, w_ref, b_ref, o_ref):
    acc = jax.lax.dot_general(
        x_ref[...], w_ref[...], (((1,), (1,)), ((), ())),
        preferred_element_type=jnp.float32)
    o_ref[...] = acc + b_ref[...]


def _topk_kernel(x_ref, h_ref, l0_ref, y_ref, cand_ref):
    nch = y_ref.shape[1]
    x2 = x_ref[...]
    y_ref[...] = x2.reshape(y_ref.shape)
    neg = jnp.float32(-jnp.inf)
    for r in range(R_CAND):
        y = y_ref[...]
        cm = jnp.max(y, axis=2)
        cand_ref[:, r * nch:(r + 1) * nch] = cm
        y_ref[...] = jnp.where(y == cm[:, :, None], neg, y)

    def sel_body(i, m):
        c = cand_ref[...]
        m = jnp.max(c, axis=1, keepdims=True)
        cand_ref[...] = jnp.where(c == m, neg, c)
        return m

    t = jax.lax.fori_loop(0, K_TOP, sel_body,
                          jnp.zeros((x2.shape[0], 1), jnp.float32))
    cnt = jnp.sum((x2 >= t).astype(jnp.float32), axis=1, keepdims=True)
    bad = cnt != jnp.float32(K_TOP)

    def fb_body(i, m):
        yy = y_ref[...]
        m = jnp.max(jnp.max(yy, axis=2), axis=1, keepdims=True)
        y_ref[...] = jnp.where(yy == m[:, :, None], neg, yy)
        return m

    @pl.when(jnp.any(bad))
    def _():
        y_ref[...] = x2.reshape(y_ref.shape)
        t2 = jax.lax.fori_loop(0, K_TOP, fb_body,
                               jnp.zeros((x2.shape[0], 1), jnp.float32))
        cand_ref[:, 0:1] = jnp.where(bad, t2, t)

    @pl.when(jnp.logical_not(jnp.any(bad)))
    def _():
        cand_ref[:, 0:1] = t

    tf = cand_ref[:, 0:1]
    mask = x2 >= tf
    h_ref[...] = jnp.where(mask, jnp.maximum(x2, 0.0), 0.0)
    l0_ref[...] = jnp.sum(
        jnp.logical_and(mask, x2 > 0).astype(jnp.float32),
        axis=1, keepdims=True)


def _dec_kernel(h_ref, w_ref, b_ref, y_ref, o_ref, lr_ref, *, nk):
    k = pl.program_id(1)

    @pl.when(k == 0)
    def _():
        o_ref[...] = jnp.zeros_like(o_ref)

    o_ref[...] += jax.lax.dot_general(
        h_ref[...], w_ref[...], (((1,), (1,)), ((), ())),
        preferred_element_type=jnp.float32)

    @pl.when(k == nk - 1)
    def _():
        pred = o_ref[...] + b_ref[...]
        o_ref[...] = pred
        d = pred - y_ref[...]
        lr_ref[...] = jnp.sum(d * d, axis=1, keepdims=True)


def kernel(mlp_input, mlp_output, W_enc, b_enc, W_dec, b_dec):
    n_tok, d_in = mlp_input.shape
    d_hid = W_enc.shape[0]
    d_out = W_dec.shape[0]
    nch = d_hid // CH

    br1 = min(BR1, n_tok)
    bh1 = min(BH1, d_hid)
    pre = pl.pallas_call(
        _enc_kernel,
        grid=(d_hid // bh1, n_tok // br1),
        in_specs=[
            pl.BlockSpec((br1, d_in), lambda h, r: (r, 0)),
            pl.BlockSpec((bh1, d_in), lambda h, r: (h, 0)),
            pl.BlockSpec((1, bh1), lambda h, r: (0, h)),
        ],
        out_specs=pl.BlockSpec((br1, bh1), lambda h, r: (r, h)),
        out_shape=jax.ShapeDtypeStruct((n_tok, d_hid), jnp.float32),
        compiler_params=pltpu.CompilerParams(
            dimension_semantics=("arbitrary", "arbitrary")),
    )(mlp_input, W_enc, b_enc.reshape(1, d_hid))

    br2 = min(BR2, n_tok)
    hidden, l0_rows = pl.pallas_call(
        _topk_kernel,
        grid=(n_tok // br2,),
        in_specs=[pl.BlockSpec((br2, d_hid), lambda i: (i, 0))],
        out_specs=[
            pl.BlockSpec((br2, d_hid), lambda i: (i, 0)),
            pl.BlockSpec((br2, 1), lambda i: (i, 0)),
        ],
        out_shape=[
            jax.ShapeDtypeStruct((n_tok, d_hid), jnp.float32),
            jax.ShapeDtypeStruct((n_tok, 1), jnp.float32),
        ],
        scratch_shapes=[
            pltpu.VMEM((br2, nch, CH), jnp.float32),
            pltpu.VMEM((br2, nch * R_CAND), jnp.float32),
        ],
        compiler_params=pltpu.CompilerParams(
            dimension_semantics=("arbitrary",)),
    )(pre)

    br3 = min(BR3, n_tok)
    bk3 = min(BK3, d_hid)
    nk = d_hid // bk3
    import functools
    predicted, loss_rows = pl.pallas_call(
        functools.partial(_dec_kernel, nk=nk),
        grid=(n_tok // br3, nk),
        in_specs=[
            pl.BlockSpec((br3, bk3), lambda r, k: (r, k)),
            pl.BlockSpec((d_out, bk3), lambda r, k: (0, k)),
            pl.BlockSpec((1, d_out), lambda r, k: (0, 0)),
            pl.BlockSpec((br3, d_out), lambda r, k: (r, 0)),
        ],
        out_specs=[
            pl.BlockSpec((br3, d_out), lambda r, k: (r, 0)),
            pl.BlockSpec((br3, 1), lambda r, k: (r, 0)),
        ],
        out_shape=[
            jax.ShapeDtypeStruct((n_tok, d_out), jnp.float32),
            jax.ShapeDtypeStruct((n_tok, 1), jnp.float32),
        ],
        compiler_params=pltpu.CompilerParams(
            dimension_semantics=("arbitrary", "arbitrary")),
    )(hidden, W_dec, b_dec.reshape(1, d_out), mlp_output)

    reconstruction_loss = jnp.sum(loss_rows) / jnp.float32(n_tok * d_out)
    l0 = jnp.sum(l0_rows) / jnp.float32(n_tok)
    sparsity_loss = jnp.asarray(0.0, dtype=jnp.float32)
    loss = reconstruction_loss
    return (predicted, hidden, loss, reconstruction_loss, sparsity_loss, l0)
